# baseline (speedup 1.0000x reference)
import functools

import jax
import jax.numpy as jnp
from jax import lax
from jax.experimental import pallas as pl
from jax.experimental.pallas import tpu as pltpu
from jax.experimental.pallas import tpu_sc as plsc

_EMBED = 64
_NC = 2
_NS = 16
_NW = _NC * _NS

_IDXROW = 128
_SUB = 2
_CH = _IDXROW * _SUB
_NBUF = 4
_LEAD = 2


def _emb_body(idx_hbm, table_hbm, out_hbm, idx_v, rows_v, gsem, ssem):
    n_rows_w = idx_v.shape[0]
    n_chunks = n_rows_w // _SUB
    wid = lax.axis_index("s") * _NC + lax.axis_index("c")
    base = wid * (n_rows_w * _IDXROW)

    pltpu.sync_copy(idx_hbm.at[wid], idx_v)

    def gather_start(b, c):
        for j in range(_SUB):
            pltpu.async_copy(
                table_hbm.at[idx_v.at[c * _SUB + j]],
                rows_v.at[b, pl.ds(j * _IDXROW, _IDXROW)],
                gsem.at[b],
            )

    def gather_wait(b):
        for j in range(_SUB):
            pltpu.make_async_copy(
                table_hbm.at[idx_v.at[0]],
                rows_v.at[b, pl.ds(j * _IDXROW, _IDXROW)],
                gsem.at[b],
            ).wait()

    def store_start(b, c):
        pltpu.async_copy(
            rows_v.at[b], out_hbm.at[pl.ds(base + c * _CH, _CH)], ssem.at[b]
        )

    def store_wait(b):
        pltpu.make_async_copy(
            rows_v.at[b], out_hbm.at[pl.ds(0, _CH)], ssem.at[b]
        ).wait()

    for c in range(_LEAD):
        gather_start(c % _NBUF, c)
    for c in range(_NBUF - _LEAD):
        gather_start((c + _LEAD) % _NBUF, c + _LEAD)
        gather_wait(c % _NBUF)
        store_start(c % _NBUF, c)

    lo, hi = _NBUF - _LEAD, n_chunks - _LEAD

    @pl.loop(lo, lo + ((hi - lo) // _NBUF) * _NBUF, step=_NBUF)
    def _steady(c0):
        for j in range(_NBUF):
            c = c0 + j
            bf = (lo + j + _LEAD) % _NBUF
            store_wait(bf)
            gather_start(bf, c + _LEAD)
            b = (lo + j) % _NBUF
            gather_wait(b)
            store_start(b, c)

    for c in range(lo + ((hi - lo) // _NBUF) * _NBUF, n_chunks):
        if c < hi:
            bf = (c + _LEAD) % _NBUF
            store_wait(bf)
            gather_start(bf, c + _LEAD)
        gather_wait(c % _NBUF)
        store_start(c % _NBUF, c)
    for b in range(_NBUF):
        store_wait(b)


@jax.jit
def kernel(x, word_embedding):
    batch, seq = x.shape
    vocab, embed = word_embedding.shape
    total = batch * seq
    n_rows_w = total // (_NW * _IDXROW)
    idx = x.reshape(_NW, n_rows_w, _IDXROW).astype(jnp.int32)

    mesh = plsc.VectorSubcoreMesh(core_axis_name="c", subcore_axis_name="s")
    grab = pl.kernel(
        _emb_body,
        out_type=jax.ShapeDtypeStruct((total, embed), jnp.float32),
        mesh=mesh,
        scratch_types=[
            pltpu.VMEM((n_rows_w, _IDXROW), jnp.int32),
            pltpu.VMEM((_NBUF, _CH, embed), jnp.float32),
            pltpu.SemaphoreType.DMA((_NBUF,)),
            pltpu.SemaphoreType.DMA((_NBUF,)),
        ],
        compiler_params=pltpu.CompilerParams(use_tc_tiling_on_sc=False),
    )
    out = grab(idx, word_embedding)
    return out.reshape(batch, seq, embed)


# speedup vs baseline: 1.0012x; 1.0012x over previous
import jax
import jax.numpy as jnp
from jax import lax
from jax.experimental import pallas as pl
from jax.experimental.pallas import tpu as pltpu
from jax.experimental.pallas import tpu_sc as plsc

_EMBED = 64
_NC = 2
_NS = 16
_NW = _NC * _NS

_IDXROW = 128
_SUB = 2
_CH = _IDXROW * _SUB
_GBUF = 4
_GLEAD = 2

_BT = 128
_TBUF = 4
_TLEAD = 2


def _gather_body(idx_hbm, table_hbm, out_hbm, idx_v, rows_v, gsem, ssem):
    n_rows_w = idx_v.shape[0]
    n_chunks = n_rows_w // _SUB
    wid = lax.axis_index("s") * _NC + lax.axis_index("c")
    base = wid * (n_rows_w * _IDXROW)

    pltpu.sync_copy(idx_hbm.at[wid], idx_v)

    def gather_start(b, c):
        for j in range(_SUB):
            pltpu.async_copy(
                table_hbm.at[idx_v.at[c * _SUB + j]],
                rows_v.at[b, pl.ds(j * _IDXROW, _IDXROW)],
                gsem.at[b],
            )

    def gather_wait(b):
        for j in range(_SUB):
            pltpu.make_async_copy(
                table_hbm.at[idx_v.at[0]],
                rows_v.at[b, pl.ds(j * _IDXROW, _IDXROW)],
                gsem.at[b],
            ).wait()

    def store_start(b, c):
        pltpu.async_copy(
            rows_v.at[b], out_hbm.at[pl.ds(base + c * _CH, _CH)], ssem.at[b]
        )

    def store_wait(b):
        pltpu.make_async_copy(
            rows_v.at[b], out_hbm.at[pl.ds(0, _CH)], ssem.at[b]
        ).wait()

    for c in range(_GLEAD):
        gather_start(c % _GBUF, c)
    for c in range(_GBUF - _GLEAD):
        gather_start((c + _GLEAD) % _GBUF, c + _GLEAD)
        gather_wait(c % _GBUF)
        store_start(c % _GBUF, c)

    lo, hi = _GBUF - _GLEAD, n_chunks - _GLEAD

    @pl.loop(lo, lo + ((hi - lo) // _GBUF) * _GBUF, step=_GBUF)
    def _steady(c0):
        for j in range(_GBUF):
            c = c0 + j
            bf = (lo + j + _GLEAD) % _GBUF
            store_wait(bf)
            gather_start(bf, c + _GLEAD)
            b = (lo + j) % _GBUF
            gather_wait(b)
            store_start(b, c)

    for c in range(lo + ((hi - lo) // _GBUF) * _GBUF, n_chunks):
        if c < hi:
            bf = (c + _GLEAD) % _GBUF
            store_wait(bf)
            gather_start(bf, c + _GLEAD)
        gather_wait(c % _GBUF)
        store_start(c % _GBUF, c)
    for b in range(_GBUF):
        store_wait(b)


def kernel(x, word_embedding):
    batch, seq = x.shape
    vocab, embed = word_embedding.shape
    total = batch * seq
    n_rows_w = total // (_NW * _IDXROW)
    n_btiles = batch // _BT
    idx = x.reshape(_NW, n_rows_w, _IDXROW).astype(jnp.int32)

    mesh = plsc.VectorSubcoreMesh(core_axis_name="c", subcore_axis_name="s")
    grab = pl.kernel(
        _gather_body,
        out_type=jax.ShapeDtypeStruct((total, embed), jnp.float32),
        mesh=mesh,
        scratch_types=[
            pltpu.VMEM((n_rows_w, _IDXROW), jnp.int32),
            pltpu.VMEM((_GBUF, _CH, embed), jnp.float32),
            pltpu.SemaphoreType.DMA((_GBUF,)),
            pltpu.SemaphoreType.DMA((_GBUF,)),
        ],
        compiler_params=pltpu.CompilerParams(use_tc_tiling_on_sc=False),
    )
    y2 = grab(idx, word_embedding)
    return y2.reshape(batch, seq, embed)
